# baseline (speedup 1.0000x reference)
import functools

import jax
import jax.numpy as jnp
from jax import lax
from jax.experimental import pallas as pl
from jax.experimental.pallas import tpu as pltpu
from jax.experimental.pallas import tpu_sc as plsc

FIELDS = 26
DIM = 16
BATCH = 4096

_info = plsc.get_sparse_core_info()
NC = _info.num_cores
NS = _info.num_subcores
NW = NC * NS
ROWS_PER_W = BATCH * FIELDS // NW
CHUNK = 4 * FIELDS
NCHUNK = ROWS_PER_W // CHUNK


def _make_sc_kernel():
    mesh = plsc.VectorSubcoreMesh(core_axis_name="c", subcore_axis_name="s")

    @functools.partial(
        pl.kernel,
        mesh=mesh,
        compiler_params=pltpu.CompilerParams(use_tc_tiling_on_sc=False),
        out_type=jax.ShapeDtypeStruct((NW, NCHUNK, CHUNK, DIM), jnp.float32),
        scratch_types=[
            pltpu.VMEM((NCHUNK, CHUNK), jnp.int32),
            pltpu.VMEM((NCHUNK, CHUNK, DIM), jnp.float32),
            pltpu.VMEM((FIELDS, DIM), jnp.float32),
            pltpu.SemaphoreType.DMA,
        ],
    )
    def sc_kernel(x_hbm, emb_hbm, arch_hbm, out_hbm, idx_v, rows_v, arch_v, sem):
        wid = lax.axis_index("s") * NC + lax.axis_index("c")

        pltpu.sync_copy(x_hbm.at[wid], idx_v)
        pltpu.sync_copy(arch_hbm, arch_v)

        def fire(j, carry):
            pltpu.async_copy(emb_hbm.at[idx_v.at[j]], rows_v.at[j], sem)
            return carry

        lax.fori_loop(0, NCHUNK, fire, 0)

        splats = [arch_v[f] for f in range(FIELDS)]

        pltpu.make_async_copy(out_hbm.at[wid], rows_v, sem).wait()

        def mul_chunk(j, carry):
            for t in range(CHUNK):
                rows_v[j, t] = rows_v[j, t] * splats[t % FIELDS]
            return carry

        lax.fori_loop(0, NCHUNK, mul_chunk, 0)

        pltpu.sync_copy(rows_v, out_hbm.at[wid])

    return sc_kernel


_sc_kernel = _make_sc_kernel()


def kernel(x, embedding, arch):
    xw = x.reshape(NW, NCHUNK, CHUNK)
    arch_b = jnp.broadcast_to(arch[:, None], (FIELDS, DIM))
    out = _sc_kernel(xw, embedding, arch_b)
    return out.reshape(BATCH, FIELDS, DIM)


# speedup vs baseline: 1.0821x; 1.0821x over previous
import functools

import jax
import jax.numpy as jnp
from jax import lax
from jax.experimental import pallas as pl
from jax.experimental.pallas import tpu as pltpu
from jax.experimental.pallas import tpu_sc as plsc

FIELDS = 26
DIM = 16
BATCH = 4096
NC = 2
NW = 32
BPW = BATCH // NW


def _make_sc_kernel():
    mesh = plsc.VectorSubcoreMesh(core_axis_name="c", subcore_axis_name="s")

    @functools.partial(
        pl.kernel,
        mesh=mesh,
        compiler_params=pltpu.CompilerParams(
            use_tc_tiling_on_sc=False, needs_layout_passes=False
        ),
        out_type=jax.ShapeDtypeStruct((FIELDS * DIM, NW, BPW), jnp.float32),
        scratch_types=[
            pltpu.VMEM((FIELDS, BPW), jnp.int32),
            pltpu.VMEM((FIELDS * BPW, DIM), jnp.float32),
            pltpu.VMEM((FIELDS * DIM, BPW), jnp.float32),
            pltpu.VMEM((FIELDS, DIM), jnp.float32),
            pltpu.SemaphoreType.DMA,
        ],
    )
    def sc_kernel(xT_hbm, emb_hbm, arch_hbm, out_hbm,
                  idx_v, rows_v, tout_v, arch_v, sem):
        wid = lax.axis_index("s") * NC + lax.axis_index("c")
        b0 = wid * BPW

        pltpu.sync_copy(xT_hbm.at[:, pl.ds(b0, BPW)], idx_v)
        pltpu.sync_copy(arch_hbm, arch_v)

        def fire(f, carry):
            pltpu.async_copy(
                emb_hbm.at[idx_v.at[f]], rows_v.at[pl.ds(f * BPW, BPW)], sem
            )
            return carry

        lax.fori_loop(0, FIELDS, fire, 0)

        lane = lax.iota(jnp.int32, 16)

        pltpu.make_async_copy(out_hbm.at[:, wid], rows_v, sem).wait()

        def trans_f(f, carry):
            splat = arch_v[f]
            fd_idx = f * DIM + lane

            def trans_b(b, carry2):
                row = rows_v[f * BPW + b] * splat
                plsc.store_scatter(
                    tout_v, [fd_idx, jnp.full((16,), b, jnp.int32)], row
                )
                return carry2

            lax.fori_loop(0, BPW, trans_b, 0)
            return carry

        lax.fori_loop(0, FIELDS, trans_f, 0)
        pltpu.sync_copy(tout_v, out_hbm.at[:, wid])

    return sc_kernel


_sc_kernel = _make_sc_kernel()


def kernel(x, embedding, arch):
    xT = x.T
    arch_b = jnp.broadcast_to(arch[:, None], (FIELDS, DIM))
    out = _sc_kernel(xT, embedding, arch_b)
    return out.reshape(FIELDS, DIM, BATCH).transpose(2, 0, 1)
